# baseline (speedup 1.0000x reference)
import functools

import jax
import jax.numpy as jnp
from jax import lax
from jax.experimental import pallas as pl
from jax.experimental.pallas import tpu as pltpu
from jax.experimental.pallas import tpu_sc as plsc

CHANNELS = 128
HEADS = 8
HP = 16
OUT_CHANNELS = 1
N_SEED = 512
ROW_BLOCK = 512

CDST = 48
NCHUNK = 209
NDPAD = NCHUNK * CDST
NWORK = 32
KMAX = 7
EBATCH = 128
OFFLEN = 320
ACOLS = 128
NPROJ = 10240

def _vsel(vec, lanes):
    return jnp.take_along_axis(vec, lanes, axis=0, mode="promise_in_bounds")


def _sc_edge_body(x_hbm, as_hbm, ad_hbm, srcs_hbm, dsts_hbm, off_hbm,
                  agg_hbm, den_hbm,
                  agg_acc, den_acc, x_buf, as_buf, ad_buf, src_buf, dst_buf,
                  off_buf, sem0, sem1):
    sems = (sem0, sem1)
    cid = lax.axis_index("c")
    sid = lax.axis_index("s")
    w = sid * 2 + cid

    pltpu.sync_copy(off_hbm, off_buf)
    iota = lax.iota(jnp.int32, 16)
    hpat = jnp.bitwise_and(iota, 7)
    hi_half = (iota >= 8).astype(jnp.int32)
    lo_mask = iota < 8
    hi_mask = iota >= 8

    def get_off(c):
        vec = off_buf[pl.ds((c // 16) * 16, 16)]
        b = _vsel(vec, jnp.full((16,), jnp.bitwise_and(c, 15), jnp.int32))
        return jnp.max(b)

    def run_chunk(c):
        estart = get_off(c)
        eend = get_off(c + 1)
        base = (estart // 16) * 16
        nb = (eend - base + EBATCH - 1) // EBATCH

        def base_of(b):
            return base + b * EBATCH

        def stage_idx(b, ph):
            g = base_of(b)
            pltpu.sync_copy(srcs_hbm.at[pl.ds(g, EBATCH)], src_buf.at[ph])
            pltpu.sync_copy(dsts_hbm.at[pl.ds(g, EBATCH)], dst_buf.at[ph])

        def fire_gathers(ph):
            pltpu.async_copy(
                as_hbm.at[src_buf.at[ph]], as_buf.at[ph], sems[ph])
            pltpu.async_copy(
                x_hbm.at[src_buf.at[ph]], x_buf.at[ph], sems[ph])

        def wait_gathers(ph):
            pltpu.make_async_copy(
                as_hbm.at[src_buf.at[ph]], as_buf.at[ph], sems[ph]).wait()
            pltpu.make_async_copy(
                x_hbm.at[src_buf.at[ph]], x_buf.at[ph], sems[ph]).wait()

        @pl.when(nb > 0)
        def _():
            stage_idx(0, 0)
            fire_gathers(0)

        def zero_body(i, _):
            for j in range(CHANNELS * HEADS // 16):
                agg_acc[i, pl.ds(j * 16, 16)] = jnp.zeros((16,), jnp.float32)
            den_acc[i, pl.ds(0, 16)] = jnp.zeros((16,), jnp.float32)
            return 0
        lax.fori_loop(0, CDST, zero_body, 0)
        pltpu.sync_copy(ad_hbm.at[pl.ds(c * CDST, CDST)], ad_buf)

        def compute_batch(b, ph):
            g0 = base_of(b)

            def do_pair(p):
                dvec = dst_buf[ph, pl.ds((p // 8) * 16, 16)]
                pi = 2 * p - (p // 8) * 16
                rowsel = jnp.full((16,), pi, jnp.int32) + hi_half
                dl = _vsel(dvec, rowsel) - c * CDST
                dlc = jnp.clip(dl, 0, CDST - 1)
                eidx = (g0 + 2 * p) + hi_half
                valid = jnp.logical_and(eidx >= estart, eidx < eend)

                bsel = jnp.full((16,), 2 * p, jnp.int32) + hi_half
                av = plsc.load_gather(as_buf.at[ph], [bsel, hpat])
                adv = plsc.load_gather(ad_buf, [dlc, hpat])
                alpha = av + adv
                alpha = jnp.where(alpha >= 0, alpha, 0.2 * alpha)
                exz = jnp.where(valid, jnp.exp(alpha), 0.0)
                plsc.addupdate_scatter(den_acc, [dlc, hpat], exz, mask=lo_mask)
                plsc.addupdate_scatter(den_acc, [dlc, hpat], exz, mask=hi_mask)

                for e in range(2):
                    lane0 = jnp.full((16,), 8 * e, jnp.int32)
                    rowv = _vsel(dlc, lane0)
                    eb = 2 * p + e
                    bcs = [_vsel(exz, lane0 + h) for h in range(HEADS)]
                    for j in range(CHANNELS // 16):
                        xv = x_buf[ph, eb, pl.ds(j * 16, 16)]
                        for h in range(HEADS):
                            col = iota + (h * CHANNELS + j * 16)
                            plsc.addupdate_scatter(
                                agg_acc, [rowv, col], bcs[h] * xv)

            def pair_body(i, _):
                do_pair(i)
                return 0

            lax.fori_loop(0, EBATCH // 2, pair_body, 0)

        def pair2_body(i, _):
            for ph in range(2):
                b = 2 * i + ph

                @pl.when(b + 1 < nb)
                def _():
                    stage_idx(b + 1, 1 - ph)
                    fire_gathers(1 - ph)

                @pl.when(b < nb)
                def _():
                    wait_gathers(ph)
                    compute_batch(b, ph)
            return 0

        lax.fori_loop(0, (nb + 1) // 2, pair2_body, 0)
        pltpu.sync_copy(agg_acc, agg_hbm.at[pl.ds(c * CDST, CDST)])
        pltpu.sync_copy(den_acc, den_hbm.at[pl.ds(c * CDST, CDST)])

    def k_body(k, _):
        c = w + NWORK * k

        @pl.when(c < NCHUNK)
        def _():
            run_chunk(c)
        return 0

    lax.fori_loop(0, KMAX, k_body, 0)


@functools.partial(jax.jit, static_argnames=())
def _sc_edge(x_src, a_src, a_dst, srcs, dsts, off):
    mesh = plsc.VectorSubcoreMesh(
        core_axis_name="c", subcore_axis_name="s", num_cores=2,
        num_subcores=16)
    f = pl.kernel(
        _sc_edge_body,
        out_type=(
            jax.ShapeDtypeStruct((NDPAD, HEADS * CHANNELS), jnp.float32),
            jax.ShapeDtypeStruct((NDPAD, HP), jnp.float32),
        ),
        mesh=mesh,
        compiler_params=pltpu.CompilerParams(needs_layout_passes=False),
        scratch_types=[
            pltpu.VMEM((CDST, HEADS * CHANNELS), jnp.float32),
            pltpu.VMEM((CDST, HP), jnp.float32),
            pltpu.VMEM((2, EBATCH, CHANNELS), jnp.float32),
            pltpu.VMEM((2, EBATCH, ACOLS), jnp.float32),
            pltpu.VMEM((CDST, HP), jnp.float32),
            pltpu.VMEM((2, EBATCH), jnp.int32),
            pltpu.VMEM((2, EBATCH), jnp.int32),
            pltpu.VMEM((OFFLEN,), jnp.int32),
            pltpu.SemaphoreType.DMA,
            pltpu.SemaphoreType.DMA,
        ],
    )
    return f(x_src, a_src, a_dst, srcs, dsts, off)


def _dense_post_kernel(agg_ref, den_ref, wfold_ref, bias_ref, w_ref, b_ref,
                       out_ref):
    n = agg_ref.shape[0]
    dinv = 1.0 / jnp.maximum(den_ref[...][:, :HEADS], 1e-30)
    agg = agg_ref[...].reshape(n, HEADS, CHANNELS) * dinv[:, :, None]
    y = jnp.dot(agg.reshape(n, HEADS * CHANNELS), wfold_ref[...],
                preferred_element_type=jnp.float32)
    y = y * (1.0 / HEADS) + bias_ref[...]
    mu = jnp.mean(y, axis=-1, keepdims=True)
    var = jnp.mean((y - mu) ** 2, axis=-1, keepdims=True)
    z = (y - mu) * lax.rsqrt(var + 1e-5) * w_ref[...] + b_ref[...]
    out_ref[...] = jnp.maximum(z, 0.0)


def _dense_post(agg, den, wfold, bias, w, b, n_out):
    grid = (pl.cdiv(n_out, ROW_BLOCK),)
    return pl.pallas_call(
        _dense_post_kernel,
        grid=grid,
        in_specs=[
            pl.BlockSpec((ROW_BLOCK, HEADS * CHANNELS), lambda i: (i, 0)),
            pl.BlockSpec((ROW_BLOCK, HP), lambda i: (i, 0)),
            pl.BlockSpec((HEADS * CHANNELS, CHANNELS), lambda i: (0, 0)),
            pl.BlockSpec((CHANNELS,), lambda i: (0,)),
            pl.BlockSpec((CHANNELS,), lambda i: (0,)),
            pl.BlockSpec((CHANNELS,), lambda i: (0,)),
        ],
        out_specs=pl.BlockSpec((ROW_BLOCK, CHANNELS), lambda i: (i, 0)),
        out_shape=jax.ShapeDtypeStruct((n_out, CHANNELS), jnp.float32),
    )(agg, den, wfold, bias, w, b)


def _att_proj_kernel(xs_ref, xd_ref, vs_ref, vd_ref, as_ref, ad_ref):
    as_ref[...] = jnp.dot(xs_ref[...], vs_ref[...],
                          preferred_element_type=jnp.float32)
    ad_ref[...] = jnp.dot(xd_ref[...], vd_ref[...],
                          preferred_element_type=jnp.float32)


def _att_proj(x_src, x_dst, v_src, v_dst):
    grid = (NPROJ // ROW_BLOCK,)
    return pl.pallas_call(
        _att_proj_kernel,
        grid=grid,
        in_specs=[
            pl.BlockSpec((ROW_BLOCK, CHANNELS), lambda i: (i, 0)),
            pl.BlockSpec((ROW_BLOCK, CHANNELS), lambda i: (i, 0)),
            pl.BlockSpec((CHANNELS, ACOLS), lambda i: (0, 0)),
            pl.BlockSpec((CHANNELS, HP), lambda i: (0, 0)),
        ],
        out_specs=[
            pl.BlockSpec((ROW_BLOCK, ACOLS), lambda i: (i, 0)),
            pl.BlockSpec((ROW_BLOCK, HP), lambda i: (i, 0)),
        ],
        out_shape=[
            jax.ShapeDtypeStruct((NPROJ, ACOLS), jnp.float32),
            jax.ShapeDtypeStruct((NPROJ, HP), jnp.float32),
        ],
    )(x_src, x_dst, v_src, v_dst)


def _fold_params(p):
    wr = p['W_src'].reshape(CHANNELS, HEADS, CHANNELS)
    wdr = p['W_dst'].reshape(CHANNELS, HEADS, CHANNELS)
    v_src = (wr * p['att_src'][None]).sum(-1)
    v_dst = (wdr * p['att_dst'][None]).sum(-1)
    v_src = jnp.concatenate(
        [v_src, jnp.zeros((CHANNELS, ACOLS - HEADS), jnp.float32)], axis=1)
    v_dst = jnp.concatenate(
        [v_dst, jnp.zeros((CHANNELS, HP - HEADS), jnp.float32)], axis=1)
    wfold = wr.transpose(1, 0, 2).reshape(HEADS * CHANNELS, CHANNELS)
    return v_src, v_dst, wfold


def _prep_edges(edge_index):
    src, dst = edge_index[0], edge_index[1]
    e = src.shape[0]
    key = jnp.sort(dst * 16384 + src)
    src_s = jnp.bitwise_and(key, 16383)
    dst_s = key >> 14
    bounds = jnp.arange(NCHUNK + 1, dtype=jnp.int32) * CDST
    off = jnp.searchsorted(dst_s, bounds, side='left').astype(jnp.int32)
    off = jnp.concatenate(
        [off, jnp.full((OFFLEN - NCHUNK - 1,), e, jnp.int32)])
    zpad = jnp.zeros((2 * EBATCH + 16,), jnp.int32)
    return (jnp.concatenate([src_s, zpad]),
            jnp.concatenate([dst_s, zpad]), off)


def _pad_rows(x):
    n = x.shape[0]
    return jnp.concatenate(
        [x, jnp.zeros((NPROJ - n,) + x.shape[1:], x.dtype)])


def _conv(x_src, x_dst, edges, p, norm):
    n_dst = x_dst.shape[0]
    v_src, v_dst, wfold = _fold_params(p)
    a_src, a_dst = _att_proj(_pad_rows(x_src), _pad_rows(x_dst), v_src, v_dst)
    srcs, dsts, off = edges
    agg, den = _sc_edge(x_src, a_src, a_dst, srcs, dsts, off)
    return _dense_post(agg, den, wfold, p['bias'], norm['w'], norm['b'],
                       n_dst)


def _head_kernel(x_ref, w_ref, b_ref, out_ref):
    out_ref[...] = (
        jnp.dot(x_ref[...], w_ref[...], preferred_element_type=jnp.float32)
        + b_ref[...]
    )


def _head(x, w, b):
    return pl.pallas_call(
        _head_kernel,
        out_shape=jax.ShapeDtypeStruct((N_SEED, OUT_CHANNELS), jnp.float32),
    )(x[:N_SEED], w, b)


def kernel(x_user, x_item, edge_index_ui, edge_index_iu, params):
    edges_ui = _prep_edges(edge_index_ui)
    edges_iu = _prep_edges(edge_index_iu)
    for l in range(2):
        lp = params['layers'][l]
        npar = params['norms'][l]
        new_item = _conv(x_user, x_item, edges_ui, lp['ui'], npar['item'])
        new_user = _conv(x_item, x_user, edges_iu, lp['iu'], npar['user'])
        x_user, x_item = new_user, new_item
    return _head(x_user, params['head']['W'], params['head']['b'])


# speedup vs baseline: 1.0216x; 1.0216x over previous
import functools

import jax
import jax.numpy as jnp
from jax import lax
from jax.experimental import pallas as pl
from jax.experimental.pallas import tpu as pltpu
from jax.experimental.pallas import tpu_sc as plsc

CHANNELS = 128
HEADS = 8
HP = 16
OUT_CHANNELS = 1
N_SEED = 512
ROW_BLOCK = 512

CDST = 32
NCHUNK = 313
NDPAD = NCHUNK * CDST
NWORK = 32
KMAX = 10
EBATCH = 128
OFFLEN = 320
ACOLS = 128
NPROJ = 10240

def _vsel(vec, lanes):
    return jnp.take_along_axis(vec, lanes, axis=0, mode="promise_in_bounds")


def _sc_edge_body(x_hbm, as_hbm, ad_hbm, srcs_hbm, dsts_hbm, off_hbm,
                  agg_hbm, den_hbm,
                  agg_acc, den_acc, x_buf, as_buf, ad_buf, src_buf, dst_buf,
                  off_buf, sem0, sem1):
    sems = (sem0, sem1)
    cid = lax.axis_index("c")
    sid = lax.axis_index("s")
    w = sid * 2 + cid

    pltpu.sync_copy(off_hbm, off_buf)
    iota = lax.iota(jnp.int32, 16)
    hpat = jnp.bitwise_and(iota, 7)
    hi_half = (iota >= 8).astype(jnp.int32)
    lo_mask = iota < 8
    hi_mask = iota >= 8

    def get_off(c):
        vec = off_buf[pl.ds((c // 16) * 16, 16)]
        b = _vsel(vec, jnp.full((16,), jnp.bitwise_and(c, 15), jnp.int32))
        return jnp.max(b)

    def run_chunk(c):
        estart = get_off(c)
        eend = get_off(c + 1)
        base = (estart // 16) * 16
        nb = (eend - base + EBATCH - 1) // EBATCH

        def base_of(b):
            return base + b * EBATCH

        def stage_idx(b, ph):
            g = base_of(b)
            pltpu.sync_copy(srcs_hbm.at[pl.ds(g, EBATCH)], src_buf.at[ph])
            pltpu.sync_copy(dsts_hbm.at[pl.ds(g, EBATCH)], dst_buf.at[ph])

        def fire_gathers(ph):
            pltpu.async_copy(
                as_hbm.at[src_buf.at[ph]], as_buf.at[ph], sems[ph])
            pltpu.async_copy(
                x_hbm.at[src_buf.at[ph]], x_buf.at[ph], sems[ph])

        def wait_gathers(ph):
            pltpu.make_async_copy(
                as_hbm.at[src_buf.at[ph]], as_buf.at[ph], sems[ph]).wait()
            pltpu.make_async_copy(
                x_hbm.at[src_buf.at[ph]], x_buf.at[ph], sems[ph]).wait()

        @pl.when(nb > 0)
        def _():
            stage_idx(0, 0)
            fire_gathers(0)

        def zero_body(i, _):
            for j in range(CHANNELS * HEADS // 16):
                agg_acc[i, pl.ds(j * 16, 16)] = jnp.zeros((16,), jnp.float32)
            den_acc[i, pl.ds(0, 16)] = jnp.zeros((16,), jnp.float32)
            return 0
        lax.fori_loop(0, CDST, zero_body, 0)
        pltpu.sync_copy(ad_hbm.at[pl.ds(c * CDST, CDST)], ad_buf)

        def compute_batch(b, ph):
            g0 = base_of(b)

            def do_pair(p):
                dvec = dst_buf[ph, pl.ds((p // 8) * 16, 16)]
                pi = 2 * p - (p // 8) * 16
                rowsel = jnp.full((16,), pi, jnp.int32) + hi_half
                dl = _vsel(dvec, rowsel) - c * CDST
                dlc = jnp.clip(dl, 0, CDST - 1)
                eidx = (g0 + 2 * p) + hi_half
                valid = jnp.logical_and(eidx >= estart, eidx < eend)

                bsel = jnp.full((16,), 2 * p, jnp.int32) + hi_half
                av = plsc.load_gather(as_buf.at[ph], [bsel, hpat])
                adv = plsc.load_gather(ad_buf, [dlc, hpat])
                alpha = av + adv
                alpha = jnp.where(alpha >= 0, alpha, 0.2 * alpha)
                exz = jnp.where(valid, jnp.exp(alpha), 0.0)
                plsc.addupdate_scatter(den_acc, [dlc, hpat], exz, mask=lo_mask)
                plsc.addupdate_scatter(den_acc, [dlc, hpat], exz, mask=hi_mask)

                for e in range(2):
                    lane0 = jnp.full((16,), 8 * e, jnp.int32)
                    rowv = _vsel(dlc, lane0)
                    eb = 2 * p + e
                    bcs = [_vsel(exz, lane0 + h) for h in range(HEADS)]
                    for j in range(CHANNELS // 16):
                        xv = x_buf[ph, eb, pl.ds(j * 16, 16)]
                        for h in range(HEADS):
                            col = iota + (h * CHANNELS + j * 16)
                            plsc.addupdate_scatter(
                                agg_acc, [rowv, col], bcs[h] * xv)

            def pair_body(i, _):
                do_pair(i)
                return 0

            lax.fori_loop(0, EBATCH // 2, pair_body, 0)

        def pair2_body(i, _):
            for ph in range(2):
                b = 2 * i + ph

                @pl.when(b + 1 < nb)
                def _():
                    stage_idx(b + 1, 1 - ph)
                    fire_gathers(1 - ph)

                @pl.when(b < nb)
                def _():
                    wait_gathers(ph)
                    compute_batch(b, ph)
            return 0

        lax.fori_loop(0, (nb + 1) // 2, pair2_body, 0)
        pltpu.sync_copy(agg_acc, agg_hbm.at[pl.ds(c * CDST, CDST)])
        pltpu.sync_copy(den_acc, den_hbm.at[pl.ds(c * CDST, CDST)])

    def k_body(k, _):
        c = w + NWORK * k

        @pl.when(c < NCHUNK)
        def _():
            run_chunk(c)
        return 0

    lax.fori_loop(0, KMAX, k_body, 0)


@functools.partial(jax.jit, static_argnames=())
def _sc_edge(x_src, a_src, a_dst, srcs, dsts, off):
    mesh = plsc.VectorSubcoreMesh(
        core_axis_name="c", subcore_axis_name="s", num_cores=2,
        num_subcores=16)
    f = pl.kernel(
        _sc_edge_body,
        out_type=(
            jax.ShapeDtypeStruct((NDPAD, HEADS * CHANNELS), jnp.float32),
            jax.ShapeDtypeStruct((NDPAD, HP), jnp.float32),
        ),
        mesh=mesh,
        compiler_params=pltpu.CompilerParams(needs_layout_passes=False),
        scratch_types=[
            pltpu.VMEM((CDST, HEADS * CHANNELS), jnp.float32),
            pltpu.VMEM((CDST, HP), jnp.float32),
            pltpu.VMEM((2, EBATCH, CHANNELS), jnp.float32),
            pltpu.VMEM((2, EBATCH, ACOLS), jnp.float32),
            pltpu.VMEM((CDST, HP), jnp.float32),
            pltpu.VMEM((2, EBATCH), jnp.int32),
            pltpu.VMEM((2, EBATCH), jnp.int32),
            pltpu.VMEM((OFFLEN,), jnp.int32),
            pltpu.SemaphoreType.DMA,
            pltpu.SemaphoreType.DMA,
        ],
    )
    return f(x_src, a_src, a_dst, srcs, dsts, off)


def _dense_post_kernel(agg_ref, den_ref, wfold_ref, bias_ref, w_ref, b_ref,
                       out_ref):
    n = agg_ref.shape[0]
    dinv = 1.0 / jnp.maximum(den_ref[...][:, :HEADS], 1e-30)
    agg = agg_ref[...].reshape(n, HEADS, CHANNELS) * dinv[:, :, None]
    y = jnp.dot(agg.reshape(n, HEADS * CHANNELS), wfold_ref[...],
                preferred_element_type=jnp.float32)
    y = y * (1.0 / HEADS) + bias_ref[...]
    mu = jnp.mean(y, axis=-1, keepdims=True)
    var = jnp.mean((y - mu) ** 2, axis=-1, keepdims=True)
    z = (y - mu) * lax.rsqrt(var + 1e-5) * w_ref[...] + b_ref[...]
    out_ref[...] = jnp.maximum(z, 0.0)


def _dense_post(agg, den, wfold, bias, w, b, n_out):
    grid = (pl.cdiv(n_out, ROW_BLOCK),)
    return pl.pallas_call(
        _dense_post_kernel,
        grid=grid,
        in_specs=[
            pl.BlockSpec((ROW_BLOCK, HEADS * CHANNELS), lambda i: (i, 0)),
            pl.BlockSpec((ROW_BLOCK, HP), lambda i: (i, 0)),
            pl.BlockSpec((HEADS * CHANNELS, CHANNELS), lambda i: (0, 0)),
            pl.BlockSpec((CHANNELS,), lambda i: (0,)),
            pl.BlockSpec((CHANNELS,), lambda i: (0,)),
            pl.BlockSpec((CHANNELS,), lambda i: (0,)),
        ],
        out_specs=pl.BlockSpec((ROW_BLOCK, CHANNELS), lambda i: (i, 0)),
        out_shape=jax.ShapeDtypeStruct((n_out, CHANNELS), jnp.float32),
    )(agg, den, wfold, bias, w, b)


def _att_proj_kernel(xs_ref, xd_ref, vs_ref, vd_ref, as_ref, ad_ref):
    as_ref[...] = jnp.dot(xs_ref[...], vs_ref[...],
                          preferred_element_type=jnp.float32)
    ad_ref[...] = jnp.dot(xd_ref[...], vd_ref[...],
                          preferred_element_type=jnp.float32)


def _att_proj(x_src, x_dst, v_src, v_dst):
    grid = (NPROJ // ROW_BLOCK,)
    return pl.pallas_call(
        _att_proj_kernel,
        grid=grid,
        in_specs=[
            pl.BlockSpec((ROW_BLOCK, CHANNELS), lambda i: (i, 0)),
            pl.BlockSpec((ROW_BLOCK, CHANNELS), lambda i: (i, 0)),
            pl.BlockSpec((CHANNELS, ACOLS), lambda i: (0, 0)),
            pl.BlockSpec((CHANNELS, HP), lambda i: (0, 0)),
        ],
        out_specs=[
            pl.BlockSpec((ROW_BLOCK, ACOLS), lambda i: (i, 0)),
            pl.BlockSpec((ROW_BLOCK, HP), lambda i: (i, 0)),
        ],
        out_shape=[
            jax.ShapeDtypeStruct((NPROJ, ACOLS), jnp.float32),
            jax.ShapeDtypeStruct((NPROJ, HP), jnp.float32),
        ],
    )(x_src, x_dst, v_src, v_dst)


def _fold_params(p):
    wr = p['W_src'].reshape(CHANNELS, HEADS, CHANNELS)
    wdr = p['W_dst'].reshape(CHANNELS, HEADS, CHANNELS)
    v_src = (wr * p['att_src'][None]).sum(-1)
    v_dst = (wdr * p['att_dst'][None]).sum(-1)
    v_src = jnp.concatenate(
        [v_src, jnp.zeros((CHANNELS, ACOLS - HEADS), jnp.float32)], axis=1)
    v_dst = jnp.concatenate(
        [v_dst, jnp.zeros((CHANNELS, HP - HEADS), jnp.float32)], axis=1)
    wfold = wr.transpose(1, 0, 2).reshape(HEADS * CHANNELS, CHANNELS)
    return v_src, v_dst, wfold


def _prep_edges(edge_index):
    src, dst = edge_index[0], edge_index[1]
    e = src.shape[0]
    key = jnp.sort(dst * 16384 + src)
    src_s = jnp.bitwise_and(key, 16383)
    dst_s = key >> 14
    bounds = jnp.arange(NCHUNK + 1, dtype=jnp.int32) * CDST
    off = jnp.searchsorted(dst_s, bounds, side='left').astype(jnp.int32)
    off = jnp.concatenate(
        [off, jnp.full((OFFLEN - NCHUNK - 1,), e, jnp.int32)])
    zpad = jnp.zeros((2 * EBATCH + 16,), jnp.int32)
    return (jnp.concatenate([src_s, zpad]),
            jnp.concatenate([dst_s, zpad]), off)


def _pad_rows(x):
    n = x.shape[0]
    return jnp.concatenate(
        [x, jnp.zeros((NPROJ - n,) + x.shape[1:], x.dtype)])


def _conv(x_src, x_dst, edges, p, norm):
    n_dst = x_dst.shape[0]
    v_src, v_dst, wfold = _fold_params(p)
    a_src, a_dst = _att_proj(_pad_rows(x_src), _pad_rows(x_dst), v_src, v_dst)
    srcs, dsts, off = edges
    agg, den = _sc_edge(x_src, a_src, a_dst, srcs, dsts, off)
    return _dense_post(agg, den, wfold, p['bias'], norm['w'], norm['b'],
                       n_dst)


def _head_kernel(x_ref, w_ref, b_ref, out_ref):
    out_ref[...] = (
        jnp.dot(x_ref[...], w_ref[...], preferred_element_type=jnp.float32)
        + b_ref[...]
    )


def _head(x, w, b):
    return pl.pallas_call(
        _head_kernel,
        out_shape=jax.ShapeDtypeStruct((N_SEED, OUT_CHANNELS), jnp.float32),
    )(x[:N_SEED], w, b)


def kernel(x_user, x_item, edge_index_ui, edge_index_iu, params):
    edges_ui = _prep_edges(edge_index_ui)
    edges_iu = _prep_edges(edge_index_iu)
    for l in range(2):
        lp = params['layers'][l]
        npar = params['norms'][l]
        new_item = _conv(x_user, x_item, edges_ui, lp['ui'], npar['item'])
        new_user = _conv(x_item, x_user, edges_iu, lp['iu'], npar['user'])
        x_user, x_item = new_user, new_item
    return _head(x_user, params['head']['W'], params['head']['b'])
